# baseline (speedup 1.0000x reference)
import functools

import jax
import jax.numpy as jnp
from jax import lax
from jax.experimental import pallas as pl
from jax.experimental.pallas import tpu as pltpu
from jax.experimental.pallas import tpu_sc as plsc

N = 10000
E = 640000
NT = 32
EPT = E // NT
U = 5

_SC_PARAMS = pltpu.CompilerParams(needs_layout_passes=False)
_MESH = plsc.VectorSubcoreMesh(core_axis_name="c", subcore_axis_name="s")


def _wid():
    return lax.axis_index("s") * 2 + lax.axis_index("c")


def _zero_vmem(ref, nwords):
    z = jnp.zeros((16,), jnp.float32)

    def body(i, carry):
        ref[pl.ds(i * 16, 16)] = z
        return carry

    lax.fori_loop(0, nwords // 16, body, None)


def _edge_pipeline(hbms, bufs, sems, base, nch, cb, setup, step_chunk):
    def start(cidx, slot):
        off = base + cidx * cb
        for h, b in zip(hbms, bufs):
            pltpu.async_copy(h.at[pl.ds(off, cb)], b[slot], sems[slot])

    def wait(cidx, slot):
        off = base + cidx * cb
        for h, b in zip(hbms, bufs):
            pltpu.make_async_copy(h.at[pl.ds(off, cb)], b[slot],
                                  sems[slot]).wait()

    start(0, 0)
    setup()

    def pair(p, carry):
        c0 = 2 * p
        start(c0 + 1, 1)
        wait(c0, 0)
        step_chunk([b[0] for b in bufs], c0)

        @pl.when(c0 + 2 < nch)
        def _():
            start(c0 + 2, 0)

        wait(c0 + 1, 1)
        step_chunk([b[1] for b in bufs], c0 + 1)
        return carry

    lax.fori_loop(0, nch // 2, pair, None)


_CB1 = 2000


@functools.partial(
    pl.kernel,
    out_type=jax.ShapeDtypeStruct((NT, N), jnp.float32),
    mesh=_MESH,
    compiler_params=_SC_PARAMS,
    scratch_types=[
        pltpu.VMEM((_CB1,), jnp.int32),
        pltpu.VMEM((_CB1,), jnp.int32),
        pltpu.VMEM((_CB1,), jnp.float32),
        pltpu.VMEM((_CB1,), jnp.float32),
        pltpu.VMEM((N,), jnp.float32),
        pltpu.SemaphoreType.DMA,
        pltpu.SemaphoreType.DMA,
    ],
)
def _sc_deg(src_hbm, w_hbm, out_hbm, srcb0, srcb1, wb0, wb1, accv,
            sem0, sem1):
    wid = _wid()
    base = wid * EPT

    def step_chunk(bs, cidx):
        srcs, ws = bs

        def body(j, carry):
            for u in range(U):
                o = j * (16 * U) + u * 16
                plsc.addupdate_scatter(accv, [srcs[pl.ds(o, 16)]],
                                       ws[pl.ds(o, 16)])
            return carry

        lax.fori_loop(0, _CB1 // (16 * U), body, None)

    _edge_pipeline([src_hbm, w_hbm], [(srcb0, srcb1), (wb0, wb1)],
                   (sem0, sem1), base, EPT // _CB1, _CB1,
                   lambda: _zero_vmem(accv, N), step_chunk)
    pltpu.sync_copy(accv, out_hbm.at[wid])


@functools.partial(
    pl.kernel,
    out_type=(
        jax.ShapeDtypeStruct((E,), jnp.float32),
        jax.ShapeDtypeStruct((NT, N), jnp.float32),
    ),
    mesh=_MESH,
    compiler_params=_SC_PARAMS,
    scratch_types=[
        pltpu.VMEM((_CB1,), jnp.int32),
        pltpu.VMEM((_CB1,), jnp.int32),
        pltpu.VMEM((_CB1,), jnp.int32),
        pltpu.VMEM((_CB1,), jnp.int32),
        pltpu.VMEM((_CB1,), jnp.float32),
        pltpu.VMEM((_CB1,), jnp.float32),
        pltpu.VMEM((_CB1,), jnp.float32),
        pltpu.VMEM((N,), jnp.float32),
        pltpu.VMEM((N,), jnp.float32),
        pltpu.VMEM((N,), jnp.float32),
        pltpu.SemaphoreType.DMA,
        pltpu.SemaphoreType.DMA,
    ],
)
def _sc_norm_tx1(src_hbm, dst_hbm, w_hbm, dis_hbm, x_hbm,
                 norm_hbm, out_hbm,
                 srcb0, srcb1, dstb0, dstb1, wb0, wb1, normv,
                 disv, xv, accv, sem0, sem1):
    wid = _wid()
    base = wid * EPT

    def setup():
        pltpu.sync_copy(dis_hbm, disv)
        pltpu.sync_copy(x_hbm, xv)
        _zero_vmem(accv, N)

    def step_chunk(bs, cidx):
        srcs, dsts, ws = bs

        def body(j, carry):
            for u in range(U):
                o = j * (16 * U) + u * 16
                s16 = srcs[pl.ds(o, 16)]
                d16 = dsts[pl.ds(o, 16)]
                w16 = ws[pl.ds(o, 16)]
                dis_s = plsc.load_gather(disv, [s16])
                dis_d = plsc.load_gather(disv, [d16])
                n16 = -(dis_s * w16 * dis_d)
                normv[pl.ds(o, 16)] = n16
                xs = plsc.load_gather(xv, [s16])
                plsc.addupdate_scatter(accv, [d16], n16 * xs)
            return carry

        lax.fori_loop(0, _CB1 // (16 * U), body, None)
        pltpu.sync_copy(normv, norm_hbm.at[pl.ds(base + cidx * _CB1, _CB1)])

    _edge_pipeline([src_hbm, dst_hbm, w_hbm],
                   [(srcb0, srcb1), (dstb0, dstb1), (wb0, wb1)],
                   (sem0, sem1), base, EPT // _CB1, _CB1, setup, step_chunk)
    pltpu.sync_copy(accv, out_hbm.at[wid])


_CB2 = 4000
_U2 = 10


def _make_sc_scatter(nfeat, fg_per_tile):
    ngroups = nfeat // fg_per_tile
    nchunks = NT // ngroups
    ept = E // nchunks

    @functools.partial(
        pl.kernel,
        out_type=jax.ShapeDtypeStruct((NT, fg_per_tile * N), jnp.float32),
        mesh=_MESH,
        compiler_params=_SC_PARAMS,
        scratch_types=[
            pltpu.VMEM((_CB2,), jnp.int32),
            pltpu.VMEM((_CB2,), jnp.int32),
            pltpu.VMEM((_CB2,), jnp.int32),
            pltpu.VMEM((_CB2,), jnp.int32),
            pltpu.VMEM((_CB2,), jnp.float32),
            pltpu.VMEM((_CB2,), jnp.float32),
            pltpu.VMEM((fg_per_tile, N), jnp.float32),
            pltpu.VMEM((fg_per_tile * N,), jnp.float32),
            pltpu.SemaphoreType.DMA,
            pltpu.SemaphoreType.DMA,
        ],
    )
    def sc_scatter(src_hbm, dst_hbm, norm_hbm, g_hbm, out_hbm,
                   srcb0, srcb1, dstb0, dstb1, normb0, normb1,
                   gtv, accv, sem0, sem1):
        wid = _wid()
        fg = wid % ngroups
        ch = wid // ngroups
        base = ch * ept

        def setup():
            pltpu.sync_copy(g_hbm.at[pl.ds(fg * fg_per_tile, fg_per_tile)],
                            gtv)
            _zero_vmem(accv, fg_per_tile * N)

        def step_chunk(bs, cidx):
            srcs, dsts, norms = bs

            def body(j, carry):
                for u in range(_U2):
                    o = j * (16 * _U2) + u * 16
                    s16 = srcs[pl.ds(o, 16)]
                    d16 = dsts[pl.ds(o, 16)]
                    n16 = norms[pl.ds(o, 16)]
                    for f in range(fg_per_tile):
                        frow = jnp.full((16,), f, jnp.int32)
                        v = plsc.load_gather(gtv, [frow, s16])
                        plsc.addupdate_scatter(accv, [d16 + f * N], v * n16)
                return carry

            lax.fori_loop(0, _CB2 // (16 * _U2), body, None)

        _edge_pipeline([src_hbm, dst_hbm, norm_hbm],
                       [(srcb0, srcb1), (dstb0, dstb1), (normb0, normb1)],
                       (sem0, sem1), base, ept // _CB2, _CB2,
                       setup, step_chunk)
        pltpu.sync_copy(accv, out_hbm.at[wid])

    return sc_scatter


_sc_scatter32 = _make_sc_scatter(32, 4)
_sc_scatter16 = _make_sc_scatter(16, 4)


def _tc_full(body, out_shapes, *args):
    outs = tuple(jax.ShapeDtypeStruct(s, jnp.float32) for s in out_shapes)
    res = pl.pallas_call(
        body,
        out_shape=outs if len(outs) > 1 else outs[0],
    )(*args)
    return res


def _tc_dis_body(degp_ref, dis_ref):
    deg = jnp.sum(degp_ref[...], axis=0, keepdims=True)
    good = deg > 0
    dis_ref[...] = jnp.where(good, lax.rsqrt(jnp.where(good, deg, 1.0)), 0.0)


def _tc_l1_body(x_ref, txp_ref, w1_ref, b1_ref, w2_ref, g2_ref, a2_ref):
    tx1 = jnp.sum(txp_ref[...], axis=0, keepdims=True)
    x = x_ref[...]
    w10 = w1_ref[0]
    w11 = w1_ref[1]
    h1 = jnp.maximum(
        jnp.transpose(w10) * x + jnp.transpose(w11) * tx1
        + jnp.transpose(b1_ref[...]), 0.0)
    dn = (((0,), (0,)), ((), ()))
    g2_ref[...] = lax.dot_general(w2_ref[1], h1, dn,
                                  preferred_element_type=jnp.float32)
    a2_ref[...] = lax.dot_general(w2_ref[0], h1, dn,
                                  preferred_element_type=jnp.float32)


def _tc_l2_body(a2_ref, txp_ref, b2_ref, w3_ref, g3_ref, a3_ref):
    tx2 = jnp.sum(txp_ref[...], axis=0)
    h2 = jnp.maximum(a2_ref[...] + tx2 + jnp.transpose(b2_ref[...]), 0.0)
    dn = (((0,), (0,)), ((), ()))
    g3_ref[...] = lax.dot_general(w3_ref[1], h2, dn,
                                  preferred_element_type=jnp.float32)
    a3_ref[...] = lax.dot_general(w3_ref[0], h2, dn,
                                  preferred_element_type=jnp.float32)


def _tc_l3_body(a3_ref, txp_ref, b3_ref, h_ref):
    tx3 = jnp.sum(txp_ref[...], axis=0)
    h3 = jnp.maximum(a3_ref[...] + tx3 + jnp.transpose(b3_ref[...]), 0.0)
    h_ref[...] = jnp.transpose(h3)


_FC_BK = 3200


def _fc_body(h_ref, w1_ref, b1_ref, w2_ref, b2_ref, w3_ref, b3_ref,
             out_ref, acc_ref):
    k = pl.program_id(0)

    @pl.when(k == 0)
    def _init():
        acc_ref[...] = jnp.zeros_like(acc_ref)

    acc_ref[...] += jnp.dot(h_ref[...], w1_ref[...],
                            preferred_element_type=jnp.float32)

    @pl.when(k == pl.num_programs(0) - 1)
    def _finish():
        y = acc_ref[...] + b1_ref[...]
        y = jnp.dot(y, w2_ref[...], preferred_element_type=jnp.float32) + b2_ref[...]
        y = jnp.dot(y, w3_ref[...], preferred_element_type=jnp.float32) + b3_ref[...]
        out_ref[...] = y


def _fc_head(h_flat, fc1_w, fc1_b, fc2_w, fc2_b, fc3_w, fc3_b):
    nsteps = N * 16 // _FC_BK
    return pl.pallas_call(
        _fc_body,
        grid=(nsteps,),
        in_specs=[
            pl.BlockSpec((1, _FC_BK), lambda k: (0, k)),
            pl.BlockSpec((_FC_BK, 256), lambda k: (k, 0)),
            pl.BlockSpec((1, 256), lambda k: (0, 0)),
            pl.BlockSpec((256, 128), lambda k: (0, 0)),
            pl.BlockSpec((1, 128), lambda k: (0, 0)),
            pl.BlockSpec((128, 9), lambda k: (0, 0)),
            pl.BlockSpec((1, 9), lambda k: (0, 0)),
        ],
        out_specs=pl.BlockSpec((1, 9), lambda k: (0, 0)),
        out_shape=jax.ShapeDtypeStruct((1, 9), jnp.float32),
        scratch_shapes=[pltpu.VMEM((1, 256), jnp.float32)],
    )(h_flat, fc1_w, fc1_b.reshape(1, 256), fc2_w, fc2_b.reshape(1, 128),
      fc3_w, fc3_b.reshape(1, 9))


def kernel(x, edge_index, edge_weight, W1, b1, W2, b2, W3, b3,
           fc1_w, fc1_b, fc2_w, fc2_b, fc3_w, fc3_b):
    src = edge_index[0]
    dst = edge_index[1]
    x1 = x.reshape(N)

    degp = _sc_deg(src, edge_weight)
    dis = _tc_full(_tc_dis_body, [(1, N)], degp)
    norm, tx1p = _sc_norm_tx1(src, dst, edge_weight, dis.reshape(N), x1)

    g2, a2 = _tc_full(_tc_l1_body, [(32, N), (32, N)],
                      x1.reshape(1, N), tx1p, W1, b1.reshape(1, 32), W2)
    tx2p = _sc_scatter32(src, dst, norm, g2)
    g3, a3 = _tc_full(_tc_l2_body, [(16, N), (16, N)],
                      a2, tx2p.reshape(4, 32, N), b2.reshape(1, 32), W3)
    tx3p = _sc_scatter16(src, dst, norm, g3)
    h = _tc_full(_tc_l3_body, [(N, 16)],
                 a3, tx3p.reshape(8, 16, N), b3.reshape(1, 16))

    return _fc_head(h.reshape(1, N * 16),
                    fc1_w, fc1_b, fc2_w, fc2_b, fc3_w, fc3_b)


# speedup vs baseline: 1.2614x; 1.2614x over previous
import functools

import jax
import jax.numpy as jnp
from jax import lax
from jax.experimental import pallas as pl
from jax.experimental.pallas import tpu as pltpu
from jax.experimental.pallas import tpu_sc as plsc

N = 10000
E = 640000
NT = 32
EPT = E // NT
U = 5

_SC_PARAMS = pltpu.CompilerParams(needs_layout_passes=False)
_MESH = plsc.VectorSubcoreMesh(core_axis_name="c", subcore_axis_name="s")


def _wid():
    return lax.axis_index("s") * 2 + lax.axis_index("c")


def _zero_vmem(ref, nwords):
    z = jnp.zeros((16,), jnp.float32)

    def body(i, carry):
        ref[pl.ds(i * 16, 16)] = z
        return carry

    lax.fori_loop(0, nwords // 16, body, None)


def _edge_pipeline(hbms, bufs, sems, base, nch, cb, setup, step_chunk):
    def start(cidx, slot):
        off = base + cidx * cb
        for h, b in zip(hbms, bufs):
            pltpu.async_copy(h.at[pl.ds(off, cb)], b[slot], sems[slot])

    def wait(cidx, slot):
        off = base + cidx * cb
        for h, b in zip(hbms, bufs):
            pltpu.make_async_copy(h.at[pl.ds(off, cb)], b[slot],
                                  sems[slot]).wait()

    start(0, 0)
    setup()

    def pair(p, carry):
        c0 = 2 * p
        start(c0 + 1, 1)
        wait(c0, 0)
        step_chunk([b[0] for b in bufs], c0)

        @pl.when(c0 + 2 < nch)
        def _():
            start(c0 + 2, 0)

        wait(c0 + 1, 1)
        step_chunk([b[1] for b in bufs], c0 + 1)
        return carry

    lax.fori_loop(0, nch // 2, pair, None)


_CB1 = 2000


@functools.partial(
    pl.kernel,
    out_type=jax.ShapeDtypeStruct((NT, N), jnp.float32),
    mesh=_MESH,
    compiler_params=_SC_PARAMS,
    scratch_types=[
        pltpu.VMEM((_CB1,), jnp.int32),
        pltpu.VMEM((_CB1,), jnp.int32),
        pltpu.VMEM((_CB1,), jnp.float32),
        pltpu.VMEM((_CB1,), jnp.float32),
        pltpu.VMEM((N,), jnp.float32),
        pltpu.SemaphoreType.DMA,
        pltpu.SemaphoreType.DMA,
    ],
)
def _sc_deg(src_hbm, w_hbm, out_hbm, srcb0, srcb1, wb0, wb1, accv,
            sem0, sem1):
    wid = _wid()
    base = wid * EPT

    def step_chunk(bs, cidx):
        srcs, ws = bs

        def body(j, carry):
            for u in range(U):
                o = j * (16 * U) + u * 16
                plsc.addupdate_scatter(accv, [srcs[pl.ds(o, 16)]],
                                       ws[pl.ds(o, 16)])
            return carry

        lax.fori_loop(0, _CB1 // (16 * U), body, None)

    _edge_pipeline([src_hbm, w_hbm], [(srcb0, srcb1), (wb0, wb1)],
                   (sem0, sem1), base, EPT // _CB1, _CB1,
                   lambda: _zero_vmem(accv, N), step_chunk)
    pltpu.sync_copy(accv, out_hbm.at[wid])


@functools.partial(
    pl.kernel,
    out_type=(
        jax.ShapeDtypeStruct((E,), jnp.float32),
        jax.ShapeDtypeStruct((NT, N), jnp.float32),
    ),
    mesh=_MESH,
    compiler_params=_SC_PARAMS,
    scratch_types=[
        pltpu.VMEM((_CB1,), jnp.int32),
        pltpu.VMEM((_CB1,), jnp.int32),
        pltpu.VMEM((_CB1,), jnp.int32),
        pltpu.VMEM((_CB1,), jnp.int32),
        pltpu.VMEM((_CB1,), jnp.float32),
        pltpu.VMEM((_CB1,), jnp.float32),
        pltpu.VMEM((_CB1,), jnp.float32),
        pltpu.VMEM((N,), jnp.float32),
        pltpu.VMEM((N,), jnp.float32),
        pltpu.VMEM((N,), jnp.float32),
        pltpu.SemaphoreType.DMA,
        pltpu.SemaphoreType.DMA,
    ],
)
def _sc_norm_tx1(src_hbm, dst_hbm, w_hbm, dis_hbm, x_hbm,
                 norm_hbm, out_hbm,
                 srcb0, srcb1, dstb0, dstb1, wb0, wb1, normv,
                 disv, xv, accv, sem0, sem1):
    wid = _wid()
    base = wid * EPT

    def setup():
        pltpu.sync_copy(dis_hbm, disv)
        pltpu.sync_copy(x_hbm, xv)
        _zero_vmem(accv, N)

    def step_chunk(bs, cidx):
        srcs, dsts, ws = bs

        def body(j, carry):
            for u in range(U):
                o = j * (16 * U) + u * 16
                s16 = srcs[pl.ds(o, 16)]
                d16 = dsts[pl.ds(o, 16)]
                w16 = ws[pl.ds(o, 16)]
                dis_s = plsc.load_gather(disv, [s16])
                dis_d = plsc.load_gather(disv, [d16])
                n16 = -(dis_s * w16 * dis_d)
                normv[pl.ds(o, 16)] = n16
                xs = plsc.load_gather(xv, [s16])
                plsc.addupdate_scatter(accv, [d16], n16 * xs)
            return carry

        lax.fori_loop(0, _CB1 // (16 * U), body, None)
        pltpu.sync_copy(normv, norm_hbm.at[pl.ds(base + cidx * _CB1, _CB1)])

    _edge_pipeline([src_hbm, dst_hbm, w_hbm],
                   [(srcb0, srcb1), (dstb0, dstb1), (wb0, wb1)],
                   (sem0, sem1), base, EPT // _CB1, _CB1, setup, step_chunk)
    pltpu.sync_copy(accv, out_hbm.at[wid])


_CB2 = 800
_U2 = 10
_FGP = 4


def _make_sc_scatter(nfeat):
    fg_feats = 2 * _FGP
    ngroups = nfeat // fg_feats
    nchunks = NT // ngroups
    ept = E // nchunks

    @functools.partial(
        pl.kernel,
        out_type=jax.ShapeDtypeStruct((NT, fg_feats * N), jnp.float32),
        mesh=_MESH,
        compiler_params=_SC_PARAMS,
        scratch_types=[
            pltpu.VMEM((_CB2,), jnp.int32),
            pltpu.VMEM((_CB2,), jnp.int32),
            pltpu.VMEM((_CB2,), jnp.int32),
            pltpu.VMEM((_CB2,), jnp.int32),
            pltpu.VMEM((_CB2,), jnp.float32),
            pltpu.VMEM((_CB2,), jnp.float32),
            pltpu.VMEM((_FGP, N), jnp.int32),
            pltpu.VMEM((fg_feats * N,), jnp.float32),
            pltpu.SemaphoreType.DMA,
            pltpu.SemaphoreType.DMA,
        ],
    )
    def sc_scatter(src_hbm, dst_hbm, norm_hbm, g_hbm, out_hbm,
                   srcb0, srcb1, dstb0, dstb1, normb0, normb1,
                   gtv, accv, sem0, sem1):
        wid = _wid()
        fg = wid % ngroups
        ch = wid // ngroups
        base = ch * ept

        def setup():
            pltpu.sync_copy(g_hbm.at[pl.ds(fg * _FGP, _FGP)], gtv)
            _zero_vmem(accv, fg_feats * N)

        def step_chunk(bs, cidx):
            srcs, dsts, norms = bs

            def body(j, carry):
                for u in range(_U2):
                    o = j * (16 * _U2) + u * 16
                    s16 = srcs[pl.ds(o, 16)]
                    d16 = dsts[pl.ds(o, 16)]
                    n16 = norms[pl.ds(o, 16)]
                    for f in range(_FGP):
                        frow = jnp.full((16,), f, jnp.int32)
                        w = plsc.load_gather(gtv, [frow, s16])
                        lo = plsc.bitcast(lax.shift_left(w, 16), jnp.float32)
                        hi = plsc.bitcast(
                            lax.bitwise_and(w, jnp.int32(-65536)), jnp.float32)
                        plsc.addupdate_scatter(
                            accv, [d16 + (2 * f) * N], lo * n16)
                        plsc.addupdate_scatter(
                            accv, [d16 + (2 * f + 1) * N], hi * n16)
                return carry

            lax.fori_loop(0, _CB2 // (16 * _U2), body, None)

        _edge_pipeline([src_hbm, dst_hbm, norm_hbm],
                       [(srcb0, srcb1), (dstb0, dstb1), (normb0, normb1)],
                       (sem0, sem1), base, ept // _CB2, _CB2,
                       setup, step_chunk)
        pltpu.sync_copy(accv, out_hbm.at[wid])

    return sc_scatter


_sc_scatter32 = _make_sc_scatter(32)
_sc_scatter16 = _make_sc_scatter(16)


def _tc_full(body, out_shapes, *args):
    outs = tuple(
        jax.ShapeDtypeStruct(*s) if isinstance(s[0], tuple)
        else jax.ShapeDtypeStruct(s, jnp.float32)
        for s in out_shapes)
    return pl.pallas_call(
        body,
        out_shape=outs if len(outs) > 1 else outs[0],
    )(*args)


def _pack_rows(g):
    f, n = g.shape
    gb = g.astype(jnp.bfloat16).reshape(f // 2, 2, n)
    even = lax.bitcast_convert_type(gb[:, 0, :], jnp.uint16).astype(jnp.uint32)
    odd = lax.bitcast_convert_type(gb[:, 1, :], jnp.uint16).astype(jnp.uint32)
    return lax.bitcast_convert_type((odd << jnp.uint32(16)) | even, jnp.int32)


def _tc_dis_body(degp_ref, dis_ref):
    deg = jnp.sum(degp_ref[...], axis=0, keepdims=True)
    good = deg > 0
    dis_ref[...] = jnp.where(good, lax.rsqrt(jnp.where(good, deg, 1.0)), 0.0)


def _tc_l1_body(x_ref, txp_ref, w1_ref, b1_ref, w2_ref, g2_ref, a2_ref):
    tx1 = jnp.sum(txp_ref[...], axis=0, keepdims=True)
    x = x_ref[...]
    w10 = w1_ref[0]
    w11 = w1_ref[1]
    h1 = jnp.maximum(
        jnp.transpose(w10) * x + jnp.transpose(w11) * tx1
        + jnp.transpose(b1_ref[...]), 0.0)
    dn = (((0,), (0,)), ((), ()))
    g2_ref[...] = _pack_rows(lax.dot_general(
        w2_ref[1], h1, dn, preferred_element_type=jnp.float32))
    a2_ref[...] = lax.dot_general(w2_ref[0], h1, dn,
                                  preferred_element_type=jnp.float32)


def _tc_l2_body(a2_ref, txp_ref, b2_ref, w3_ref, g3_ref, a3_ref):
    tx2 = jnp.sum(txp_ref[...], axis=0)
    h2 = jnp.maximum(a2_ref[...] + tx2 + jnp.transpose(b2_ref[...]), 0.0)
    dn = (((0,), (0,)), ((), ()))
    g3_ref[...] = _pack_rows(lax.dot_general(
        w3_ref[1], h2, dn, preferred_element_type=jnp.float32))
    a3_ref[...] = lax.dot_general(w3_ref[0], h2, dn,
                                  preferred_element_type=jnp.float32)


def _tc_l3_body(a3_ref, txp_ref, b3_ref, h_ref):
    tx3 = jnp.sum(txp_ref[...], axis=0)
    h3 = jnp.maximum(a3_ref[...] + tx3 + jnp.transpose(b3_ref[...]), 0.0)
    h_ref[...] = jnp.transpose(h3)


_FC_BK = 3200


def _fc_body(h_ref, w1_ref, b1_ref, w2_ref, b2_ref, w3_ref, b3_ref,
             out_ref, acc_ref):
    k = pl.program_id(0)

    @pl.when(k == 0)
    def _init():
        acc_ref[...] = jnp.zeros_like(acc_ref)

    acc_ref[...] += jnp.dot(h_ref[...], w1_ref[...],
                            preferred_element_type=jnp.float32)

    @pl.when(k == pl.num_programs(0) - 1)
    def _finish():
        y = acc_ref[...] + b1_ref[...]
        y = jnp.dot(y, w2_ref[...], preferred_element_type=jnp.float32) + b2_ref[...]
        y = jnp.dot(y, w3_ref[...], preferred_element_type=jnp.float32) + b3_ref[...]
        out_ref[...] = y


def _fc_head(h_flat, fc1_w, fc1_b, fc2_w, fc2_b, fc3_w, fc3_b):
    nsteps = N * 16 // _FC_BK
    return pl.pallas_call(
        _fc_body,
        grid=(nsteps,),
        in_specs=[
            pl.BlockSpec((1, _FC_BK), lambda k: (0, k)),
            pl.BlockSpec((_FC_BK, 256), lambda k: (k, 0)),
            pl.BlockSpec((1, 256), lambda k: (0, 0)),
            pl.BlockSpec((256, 128), lambda k: (0, 0)),
            pl.BlockSpec((1, 128), lambda k: (0, 0)),
            pl.BlockSpec((128, 9), lambda k: (0, 0)),
            pl.BlockSpec((1, 9), lambda k: (0, 0)),
        ],
        out_specs=pl.BlockSpec((1, 9), lambda k: (0, 0)),
        out_shape=jax.ShapeDtypeStruct((1, 9), jnp.float32),
        scratch_shapes=[pltpu.VMEM((1, 256), jnp.float32)],
    )(h_flat, fc1_w, fc1_b.reshape(1, 256), fc2_w, fc2_b.reshape(1, 128),
      fc3_w, fc3_b.reshape(1, 9))


def kernel(x, edge_index, edge_weight, W1, b1, W2, b2, W3, b3,
           fc1_w, fc1_b, fc2_w, fc2_b, fc3_w, fc3_b):
    src = edge_index[0]
    dst = edge_index[1]
    x1 = x.reshape(N)

    degp = _sc_deg(src, edge_weight)
    dis = _tc_full(_tc_dis_body, [(1, N)], degp)
    norm, tx1p = _sc_norm_tx1(src, dst, edge_weight, dis.reshape(N), x1)

    g2, a2 = _tc_full(_tc_l1_body, [((16, N), jnp.int32), (32, N)],
                      x1.reshape(1, N), tx1p, W1, b1.reshape(1, 32), W2)
    tx2p = _sc_scatter32(src, dst, norm, g2)
    g3, a3 = _tc_full(_tc_l2_body, [((8, N), jnp.int32), (16, N)],
                      a2, tx2p.reshape(8, 32, N), b2.reshape(1, 32), W3)
    tx3p = _sc_scatter16(src, dst, norm, g3)
    h = _tc_full(_tc_l3_body, [(N, 16)],
                 a3, tx3p.reshape(16, 16, N), b3.reshape(1, 16))

    return _fc_head(h.reshape(1, N * 16),
                    fc1_w, fc1_b, fc2_w, fc2_b, fc3_w, fc3_b)
